# baseline (speedup 1.0000x reference)
import functools

import jax
import jax.numpy as jnp
from jax import lax
from jax.experimental import pallas as pl
from jax.experimental.pallas import tpu as pltpu
from jax.experimental.pallas import tpu_sc as plsc


_CHUNK = 640


def _sc_gather_native(embT, tailT, idx, n_full, tail):
    D, V = embT.shape
    B = idx.shape[0]
    info = plsc.get_sparse_core_info()
    nc, ns = info.num_cores, info.num_subcores
    nw = nc * ns
    n_vregs = B // 16
    rounds = (n_full + nw - 1) // nw
    mesh = plsc.VectorSubcoreMesh(core_axis_name="c", subcore_axis_name="s")

    @functools.partial(
        pl.kernel,
        mesh=mesh,
        out_type=jax.ShapeDtypeStruct((B + 1, 128), jnp.float32),
        scratch_types=[
            pltpu.VMEM((B,), jnp.int32),
            pltpu.VMEM((D, _CHUNK), jnp.float32),
            pltpu.VMEM((D, 256), jnp.float32),
            pltpu.VMEM((B, 128), jnp.float32),
            pltpu.VMEM((B,), jnp.int32),
            pltpu.VMEM((B,), jnp.int32),
            pltpu.SemaphoreType.DMA,
        ],
        compiler_params=pltpu.CompilerParams(needs_layout_passes=False),
    )
    def gather_kernel(table_hbm, tail_hbm, idx_hbm, out_hbm, idx_v, cbuf,
                      tail_buf, stage, own_v, rid_v, sem):
        wid = lax.axis_index("s") * nc + lax.axis_index("c")
        pltpu.sync_copy(idx_hbm, idx_v)
        lane = lax.iota(jnp.int32, 16)
        zeros16 = jnp.zeros((16,), jnp.int32)

        @pl.loop(0, n_vregs)
        def _init(t):
            own_v[pl.ds(t * 16, 16)] = zeros16

        def scan_chunk(buf, c0, csize):
            @pl.loop(0, n_vregs)
            def _vloop(t):
                toks = idx_v[pl.ds(t * 16, 16)]
                cols = toks - c0
                m = (cols >= 0) & (cols < csize)
                colsc = jnp.where(m, cols, 0)

                @pl.loop(0, D)
                def _floop(f):
                    fvec = jnp.full((16,), f, jnp.int32)
                    vals = plsc.load_gather(buf, [fvec, colsc], mask=m)
                    plsc.store_scatter(stage, [lane + t * 16, fvec], vals,
                                       mask=m)

                prev = own_v[pl.ds(t * 16, 16)]
                own_v[pl.ds(t * 16, 16)] = prev | m.astype(jnp.int32)

        for r in range(rounds):
            if (r + 1) * nw <= n_full:
                cid = wid + nw * r
                pltpu.sync_copy(table_hbm.at[:, pl.ds(cid * _CHUNK, _CHUNK)],
                                cbuf)
                scan_chunk(cbuf, cid * _CHUNK, _CHUNK)
            else:
                @pl.when(wid + nw * r < n_full)
                def _():
                    cid = wid + nw * r
                    pltpu.sync_copy(
                        table_hbm.at[:, pl.ds(cid * _CHUNK, _CHUNK)], cbuf)
                    scan_chunk(cbuf, cid * _CHUNK, _CHUNK)

        if tail > 0:
            @pl.when(wid == n_full - nw * (rounds - 1))
            def _():
                pltpu.sync_copy(tail_hbm, tail_buf)
                scan_chunk(tail_buf, n_full * _CHUNK, tail)

        @pl.loop(0, n_vregs)
        def _ridloop(t):
            ow = own_v[pl.ds(t * 16, 16)]
            rid_v[pl.ds(t * 16, 16)] = jnp.where(ow > 0, lane + t * 16, B)

        pltpu.async_copy(stage, out_hbm.at[rid_v], sem).wait()

    return gather_kernel(embT, tailT, idx)


def _proj_kernel(x_ref, wt_ref, b_ref, o_ref):
    x = x_ref[:, : wt_ref.shape[0]]
    o_ref[...] = lax.dot_general(
        x, wt_ref[...],
        (((1,), (0,)), ((), ())),
        preferred_element_type=jnp.float32,
    ) + b_ref[...]


def _tc_project(x_pad, Wt, b2d, v_blk):
    n_tok = x_pad.shape[0] - 1
    d, v = Wt.shape
    grid = (pl.cdiv(v, v_blk),)
    return pl.pallas_call(
        _proj_kernel,
        grid=grid,
        in_specs=[
            pl.BlockSpec((n_tok, 128), lambda i: (0, 0)),
            pl.BlockSpec((d, v_blk), lambda i: (0, i)),
            pl.BlockSpec((1, v_blk), lambda i: (0, i)),
        ],
        out_specs=pl.BlockSpec((n_tok, v_blk), lambda i: (0, i)),
        out_shape=jax.ShapeDtypeStruct((n_tok, v), jnp.float32),
        compiler_params=pltpu.CompilerParams(
            dimension_semantics=("parallel",),
        ),
    )(x_pad, Wt, b2d)


def kernel(tokens, emb, W, b):
    bsz, seq = tokens.shape
    v, d = emb.shape
    idx = tokens.reshape(bsz * seq).astype(jnp.int32)
    embT = emb.T
    n_full = v // _CHUNK
    tail = v - n_full * _CHUNK
    tailT = jnp.pad(
        lax.slice(embT, (0, n_full * _CHUNK), (d, v)),
        ((0, 0), (0, 256 - tail)),
    )
    x_pad = _sc_gather_native(embT, tailT, idx, n_full, tail)
    y = _tc_project(x_pad, W.T, b.reshape(1, v), v_blk=4096)
    return y.reshape(bsz, seq, v)
